# baseline (speedup 1.0000x reference)
import functools

import jax
import jax.numpy as jnp
from jax import lax
from jax.experimental import pallas as pl
from jax.experimental.pallas import tpu as pltpu
from jax.experimental.pallas import tpu_sc as plsc

B = 16384
ED = 64
FD = 128
HD = 128
CHUNK = 128
NC = 2
NS = 16
NW = NC * NS
K = B // (CHUNK * NW)

@functools.cache
def _make_sc_gather():
    mesh = plsc.VectorSubcoreMesh(core_axis_name="c", subcore_axis_name="s")

    @functools.partial(
        pl.kernel,
        mesh=mesh,
        out_type=[
            jax.ShapeDtypeStruct((B // CHUNK, CHUNK, ED), jnp.float32),
            jax.ShapeDtypeStruct((B // CHUNK, CHUNK, ED), jnp.float32),
        ],
        scratch_types=[
            pltpu.VMEM((K, CHUNK), jnp.int32),
            pltpu.VMEM((K, CHUNK), jnp.int32),
            pltpu.VMEM((K, CHUNK, ED), jnp.float32),
            pltpu.VMEM((K, CHUNK, ED), jnp.float32),
            pltpu.SemaphoreType.DMA,
            pltpu.SemaphoreType.DMA,
        ],
        compiler_params=pltpu.CompilerParams(use_tc_tiling_on_sc=False),
    )
    def _sc_gather(uidx_hbm, pidx_hbm, utab_hbm, ptab_hbm, uout_hbm, pout_hbm,
                   uidx_v, pidx_v, urows_v, prows_v, sem_u, sem_p):
        wid = lax.axis_index("s") * NC + lax.axis_index("c")
        base = wid * K
        pltpu.sync_copy(uidx_hbm.at[pl.ds(base, K)], uidx_v)
        pltpu.sync_copy(pidx_hbm.at[pl.ds(base, K)], pidx_v)
        copies = []
        for j in range(K):
            copies.append(pltpu.async_copy(utab_hbm.at[uidx_v.at[j]], urows_v.at[j], sem_u))
            copies.append(pltpu.async_copy(ptab_hbm.at[pidx_v.at[j]], prows_v.at[j], sem_p))
        for c in copies:
            c.wait()
        pltpu.sync_copy(urows_v, uout_hbm.at[pl.ds(base, K)])
        pltpu.sync_copy(prows_v, pout_hbm.at[pl.ds(base, K)])

    return _sc_gather


def _mlp_body(u_ref, p_ref, f_ref, w1a_ref, w1b_ref, w1c_ref, b1_ref,
              w2_ref, b2_ref, w3_ref, b3_ref, o_ref):
    dot = functools.partial(jnp.dot, preferred_element_type=jnp.float32,
                            precision=lax.Precision.HIGHEST)
    h = dot(u_ref[...], w1a_ref[...])
    h = h + dot(p_ref[...], w1b_ref[...])
    h = h + dot(f_ref[...], w1c_ref[...])
    h = jnp.maximum(h + b1_ref[...], 0.0)
    h = jnp.maximum(dot(h, w2_ref[...]) + b2_ref[...], 0.0)
    v = jnp.sum(h * w3_ref[...], axis=1) + b3_ref[0, 0]
    o_ref[0, 0, :] = 1.0 / (1.0 + jnp.exp(-v))


def _mlp(u, p, f, w1a, w1b, w1c, b1, w2, b2, w3r, b3s, bb):
    grid = B // bb
    full = lambda shape: pl.BlockSpec(shape, lambda i: (0, 0))
    return pl.pallas_call(
        _mlp_body,
        grid=(grid,),
        in_specs=[
            pl.BlockSpec((bb, ED), lambda i: (i, 0)),
            pl.BlockSpec((bb, ED), lambda i: (i, 0)),
            pl.BlockSpec((bb, FD), lambda i: (i, 0)),
            full((ED, HD)),
            full((ED, HD)),
            full((FD, HD)),
            full((1, HD)),
            full((HD, HD)),
            full((1, HD)),
            full((1, HD)),
            full((1, 1)),
        ],
        out_specs=pl.BlockSpec((1, 1, bb), lambda i: (i, 0, 0)),
        out_shape=jax.ShapeDtypeStruct((grid, 1, bb), jnp.float32),
    )(u, p, f, w1a, w1b, w1c, b1, w2, b2, w3r, b3s)


def kernel(user_indices, post_indices, features, user_table, post_table,
           W1, b1, W2, b2, W3, b3):
    uidx = user_indices.astype(jnp.int32).reshape(B // CHUNK, CHUNK)
    pidx = post_indices.astype(jnp.int32).reshape(B // CHUNK, CHUNK)
    uout, pout = _make_sc_gather()(uidx, pidx, user_table, post_table)
    u = uout.reshape(B, ED)
    p = pout.reshape(B, ED)
    out2d = _mlp(u, p, features,
                 W1[:ED], W1[ED:2 * ED], W1[2 * ED:],
                 b1.reshape(1, HD), W2, b2.reshape(1, HD),
                 W3.reshape(1, HD), b3.reshape(1, 1), 2048)
    return out2d.reshape(B)


# speedup vs baseline: 1.5401x; 1.5401x over previous
import functools

import jax
import jax.numpy as jnp
from jax import lax
from jax.experimental import pallas as pl
from jax.experimental.pallas import tpu as pltpu
from jax.experimental.pallas import tpu_sc as plsc

B = 16384
ED = 64
FD = 128
HD = 128
CHUNK = 128
NC = 2
NS = 16
NW = NC * NS
K = B // (CHUNK * NW)

RPW = B // NW
L = 16


@functools.cache
def _make_sc_gather():
    mesh = plsc.VectorSubcoreMesh(core_axis_name="c", subcore_axis_name="s")

    @functools.partial(
        pl.kernel,
        mesh=mesh,
        out_type=[
            jax.ShapeDtypeStruct((NW, RPW, ED), jnp.float32),
            jax.ShapeDtypeStruct((NW, RPW, ED), jnp.float32),
        ],
        scratch_types=[
            pltpu.VMEM((RPW,), jnp.int32),
            pltpu.VMEM((RPW,), jnp.int32),
            pltpu.VMEM((RPW, ED), jnp.float32),
            pltpu.SemaphoreType.DMA,
        ],
    )
    def _sc_gather(uidx_hbm, pidx_hbm, utab_hbm, ptab_hbm, uout_hbm, pout_hbm,
                   uidx_v, pidx_v, rows_v, sem):
        wid = lax.axis_index("s") * NC + lax.axis_index("c")
        pltpu.sync_copy(uidx_hbm.at[pl.ds(wid * RPW, RPW)], uidx_v)
        pltpu.sync_copy(pidx_hbm.at[pl.ds(wid * RPW, RPW)], pidx_v)

        def one_table(idx_v, tab_hbm, out_hbm):
            def issue(g, carry):
                vec = idx_v[pl.ds(g * L, L)]
                for lane in range(L):
                    r = vec[lane]
                    pltpu.async_copy(tab_hbm.at[pl.ds(r, 1)],
                                     rows_v.at[pl.ds(g * L + lane, 1)], sem)
                return carry

            lax.fori_loop(0, RPW // L, issue, 0)

            def drain(i, carry):
                pltpu.make_async_copy(tab_hbm.at[pl.ds(0, 1)],
                                      rows_v.at[pl.ds(i, 1)], sem).wait()
                return carry

            lax.fori_loop(0, RPW, drain, 0)
            pltpu.sync_copy(rows_v, out_hbm.at[wid])

        one_table(uidx_v, utab_hbm, uout_hbm)
        one_table(pidx_v, ptab_hbm, pout_hbm)

    return _sc_gather


def _mlp_body(u_ref, p_ref, f_ref, w1a_ref, w1b_ref, w1c_ref, b1_ref,
              w2_ref, b2_ref, w3_ref, b3_ref, o_ref):
    dot = functools.partial(jnp.dot, preferred_element_type=jnp.float32,
                            precision=lax.Precision.HIGHEST)
    h = dot(u_ref[...], w1a_ref[...])
    h = h + dot(p_ref[...], w1b_ref[...])
    h = h + dot(f_ref[...], w1c_ref[...])
    h = jnp.maximum(h + b1_ref[...], 0.0)
    h = jnp.maximum(dot(h, w2_ref[...]) + b2_ref[...], 0.0)
    v = jnp.sum(h * w3_ref[...], axis=1) + b3_ref[0, 0]
    o_ref[0, 0, :] = 1.0 / (1.0 + jnp.exp(-v))


def _mlp(u, p, f, w1a, w1b, w1c, b1, w2, b2, w3r, b3s, bb):
    grid = B // bb
    full = lambda shape: pl.BlockSpec(shape, lambda i: (0, 0))
    return pl.pallas_call(
        _mlp_body,
        grid=(grid,),
        in_specs=[
            pl.BlockSpec((bb, ED), lambda i: (i, 0)),
            pl.BlockSpec((bb, ED), lambda i: (i, 0)),
            pl.BlockSpec((bb, FD), lambda i: (i, 0)),
            full((ED, HD)),
            full((ED, HD)),
            full((FD, HD)),
            full((1, HD)),
            full((HD, HD)),
            full((1, HD)),
            full((1, HD)),
            full((1, 1)),
        ],
        out_specs=pl.BlockSpec((1, 1, bb), lambda i: (i, 0, 0)),
        out_shape=jax.ShapeDtypeStruct((grid, 1, bb), jnp.float32),
    )(u, p, f, w1a, w1b, w1c, b1, w2, b2, w3r, b3s)


def kernel(user_indices, post_indices, features, user_table, post_table,
           W1, b1, W2, b2, W3, b3):
    uidx = user_indices.astype(jnp.int32)
    pidx = post_indices.astype(jnp.int32)
    uout, pout = _make_sc_gather()(uidx, pidx, user_table, post_table)
    u = uout.reshape(B, ED)
    p = pout.reshape(B, ED)
    out2d = _mlp(u, p, features,
                 W1[:ED], W1[ED:2 * ED], W1[2 * ED:],
                 b1.reshape(1, HD), W2, b2.reshape(1, HD),
                 W3.reshape(1, HD), b3.reshape(1, 1), 2048)
    return out2d.reshape(B)


# speedup vs baseline: 2.0466x; 1.3288x over previous
import functools

import jax
import jax.numpy as jnp
from jax import lax
from jax.experimental import pallas as pl
from jax.experimental.pallas import tpu as pltpu
from jax.experimental.pallas import tpu_sc as plsc

B = 16384
ED = 64
FD = 128
HD = 128
NROWS = 1000000
NC = 2
NS = 16
NW = NC * NS
RPW = B // NW
L = 16
TBLK = 8192


def _tr_body(ut_ref, pt_ref, xu_ref, xp_ref):
    xu_ref[...] = ut_ref[...].T
    xp_ref[...] = pt_ref[...].T


def _transpose_tables(utabT, ptabT):
    grid = (NROWS + TBLK - 1) // TBLK
    return pl.pallas_call(
        _tr_body,
        grid=(grid,),
        in_specs=[
            pl.BlockSpec((ED, TBLK), lambda i: (0, i)),
            pl.BlockSpec((ED, TBLK), lambda i: (0, i)),
        ],
        out_specs=[
            pl.BlockSpec((TBLK, ED), lambda i: (i, 0)),
            pl.BlockSpec((TBLK, ED), lambda i: (i, 0)),
        ],
        out_shape=[
            jax.ShapeDtypeStruct((NROWS, ED), jnp.float32),
            jax.ShapeDtypeStruct((NROWS, ED), jnp.float32),
        ],
    )(utabT, ptabT)


@functools.cache
def _make_sc_gather():
    mesh = plsc.VectorSubcoreMesh(core_axis_name="c", subcore_axis_name="s")

    @functools.partial(
        pl.kernel,
        mesh=mesh,
        out_type=[
            jax.ShapeDtypeStruct((NW, RPW, ED), jnp.float32),
            jax.ShapeDtypeStruct((NW, RPW, ED), jnp.float32),
        ],
        scratch_types=[
            pltpu.VMEM((RPW,), jnp.int32),
            pltpu.VMEM((RPW,), jnp.int32),
            pltpu.VMEM((RPW, ED), jnp.float32),
            pltpu.SemaphoreType.DMA,
        ],
    )
    def _sc_gather(uidx_hbm, pidx_hbm, utab_hbm, ptab_hbm, uout_hbm, pout_hbm,
                   uidx_v, pidx_v, rows_v, sem):
        wid = lax.axis_index("s") * NC + lax.axis_index("c")
        pltpu.sync_copy(uidx_hbm.at[pl.ds(wid * RPW, RPW)], uidx_v)
        pltpu.sync_copy(pidx_hbm.at[pl.ds(wid * RPW, RPW)], pidx_v)

        def one_table(idx_v, tab_hbm, out_hbm):
            def issue(g, carry):
                vec = idx_v[pl.ds(g * L, L)]
                for lane in range(L):
                    r = vec[lane]
                    pltpu.async_copy(tab_hbm.at[pl.ds(r, 1)],
                                     rows_v.at[pl.ds(g * L + lane, 1)], sem)
                return carry

            lax.fori_loop(0, RPW // L, issue, 0)

            def drain(i, carry):
                pltpu.make_async_copy(tab_hbm.at[pl.ds(0, 1)],
                                      rows_v.at[pl.ds(i, 1)], sem).wait()
                return carry

            lax.fori_loop(0, RPW, drain, 0)
            pltpu.sync_copy(rows_v, out_hbm.at[wid])

        one_table(uidx_v, utab_hbm, uout_hbm)
        one_table(pidx_v, ptab_hbm, pout_hbm)

    return _sc_gather


def _mlp_body(u_ref, p_ref, f_ref, w1a_ref, w1b_ref, w1c_ref, b1_ref,
              w2_ref, b2_ref, w3_ref, b3_ref, o_ref):
    dot = functools.partial(jnp.dot, preferred_element_type=jnp.float32,
                            precision=lax.Precision.HIGHEST)
    h = dot(u_ref[...], w1a_ref[...])
    h = h + dot(p_ref[...], w1b_ref[...])
    h = h + dot(f_ref[...], w1c_ref[...])
    h = jnp.maximum(h + b1_ref[...], 0.0)
    h = jnp.maximum(dot(h, w2_ref[...]) + b2_ref[...], 0.0)
    v = jnp.sum(h * w3_ref[...], axis=1) + b3_ref[0, 0]
    o_ref[0, 0, :] = 1.0 / (1.0 + jnp.exp(-v))


def _mlp(u, p, f, w1a, w1b, w1c, b1, w2, b2, w3r, b3s, bb):
    grid = B // bb
    full = lambda shape: pl.BlockSpec(shape, lambda i: (0, 0))
    return pl.pallas_call(
        _mlp_body,
        grid=(grid,),
        in_specs=[
            pl.BlockSpec((bb, ED), lambda i: (i, 0)),
            pl.BlockSpec((bb, ED), lambda i: (i, 0)),
            pl.BlockSpec((bb, FD), lambda i: (i, 0)),
            full((ED, HD)),
            full((ED, HD)),
            full((FD, HD)),
            full((1, HD)),
            full((HD, HD)),
            full((1, HD)),
            full((1, HD)),
            full((1, 1)),
        ],
        out_specs=pl.BlockSpec((1, 1, bb), lambda i: (i, 0, 0)),
        out_shape=jax.ShapeDtypeStruct((grid, 1, bb), jnp.float32),
    )(u, p, f, w1a, w1b, w1c, b1, w2, b2, w3r, b3s)


def kernel(user_indices, post_indices, features, user_table, post_table,
           W1, b1, W2, b2, W3, b3):
    uidx = user_indices.astype(jnp.int32)
    pidx = post_indices.astype(jnp.int32)
    utab_rm, ptab_rm = _transpose_tables(user_table.T, post_table.T)
    uout, pout = _make_sc_gather()(uidx, pidx, utab_rm, ptab_rm)
    u = uout.reshape(B, ED)
    p = pout.reshape(B, ED)
    out = _mlp(u, p, features,
               W1[:ED], W1[ED:2 * ED], W1[2 * ED:],
               b1.reshape(1, HD), W2, b2.reshape(1, HD),
               W3.reshape(1, HD), b3.reshape(1, 1), 2048)
    return out.reshape(B)


# speedup vs baseline: 2.2241x; 1.0867x over previous
import functools

import jax
import jax.numpy as jnp
from jax import lax
from jax.experimental import pallas as pl
from jax.experimental.pallas import tpu as pltpu
from jax.experimental.pallas import tpu_sc as plsc

B = 16384
ED = 64
FD = 128
HD = 128
NROWS = 1000000
NC = 2
NS = 16
NW = NC * NS
RPW = B // NW
L = 16
TBLK = 16384


def _tr_body(ut_ref, pt_ref, xu_ref, xp_ref):
    xu_ref[...] = ut_ref[...].T
    xp_ref[...] = pt_ref[...].T


def _transpose_tables(utabT, ptabT):
    grid = (NROWS + TBLK - 1) // TBLK
    return pl.pallas_call(
        _tr_body,
        grid=(grid,),
        in_specs=[
            pl.BlockSpec((ED, TBLK), lambda i: (0, i)),
            pl.BlockSpec((ED, TBLK), lambda i: (0, i)),
        ],
        out_specs=[
            pl.BlockSpec((TBLK, ED), lambda i: (i, 0)),
            pl.BlockSpec((TBLK, ED), lambda i: (i, 0)),
        ],
        out_shape=[
            jax.ShapeDtypeStruct((NROWS, ED), jnp.float32),
            jax.ShapeDtypeStruct((NROWS, ED), jnp.float32),
        ],
    )(utabT, ptabT)


@functools.cache
def _make_sc_gather():
    mesh = plsc.VectorSubcoreMesh(core_axis_name="c", subcore_axis_name="s")

    @functools.partial(
        pl.kernel,
        mesh=mesh,
        out_type=[
            jax.ShapeDtypeStruct((NW, RPW, ED), jnp.float32),
            jax.ShapeDtypeStruct((NW, RPW, ED), jnp.float32),
        ],
        scratch_types=[
            pltpu.VMEM((RPW,), jnp.int32),
            pltpu.VMEM((RPW,), jnp.int32),
            pltpu.VMEM((RPW, ED), jnp.float32),
            pltpu.SemaphoreType.DMA,
        ],
    )
    def _sc_gather(uidx_hbm, pidx_hbm, utab_hbm, ptab_hbm, uout_hbm, pout_hbm,
                   uidx_v, pidx_v, rows_v, sem):
        wid = lax.axis_index("s") * NC + lax.axis_index("c")
        pltpu.sync_copy(uidx_hbm.at[pl.ds(wid * RPW, RPW)], uidx_v)
        pltpu.sync_copy(pidx_hbm.at[pl.ds(wid * RPW, RPW)], pidx_v)

        def one_table(idx_v, tab_hbm, out_hbm):
            def issue(g, carry):
                vec = idx_v[pl.ds(g * L, L)]
                for lane in range(L):
                    r = vec[lane]
                    pltpu.async_copy(tab_hbm.at[pl.ds(r, 1)],
                                     rows_v.at[pl.ds(g * L + lane, 1)], sem)
                return carry

            lax.fori_loop(0, RPW // L, issue, 0)

            def drain(i, carry):
                pltpu.make_async_copy(tab_hbm.at[pl.ds(0, 1)],
                                      rows_v.at[pl.ds(i, 1)], sem).wait()
                return carry

            lax.fori_loop(0, RPW, drain, 0)
            pltpu.sync_copy(rows_v, out_hbm.at[wid])

        one_table(uidx_v, utab_hbm, uout_hbm)
        one_table(pidx_v, ptab_hbm, pout_hbm)

    return _sc_gather


def _mlp_body(u_ref, p_ref, f_ref, w1a_ref, w1b_ref, w1c_ref, b1_ref,
              w2_ref, b2_ref, w3_ref, b3_ref, o_ref):
    dot = functools.partial(jnp.dot, preferred_element_type=jnp.float32,
                            precision=lax.Precision.DEFAULT)
    h = dot(u_ref[...], w1a_ref[...])
    h = h + dot(p_ref[...], w1b_ref[...])
    h = h + dot(f_ref[...], w1c_ref[...])
    h = jnp.maximum(h + b1_ref[...], 0.0)
    h = jnp.maximum(dot(h, w2_ref[...]) + b2_ref[...], 0.0)
    v = jnp.sum(h * w3_ref[...], axis=1) + b3_ref[0, 0]
    o_ref[0, 0, :] = 1.0 / (1.0 + jnp.exp(-v))


def _mlp(u, p, f, w1a, w1b, w1c, b1, w2, b2, w3r, b3s, bb):
    grid = B // bb
    full = lambda shape: pl.BlockSpec(shape, lambda i: (0, 0))
    return pl.pallas_call(
        _mlp_body,
        grid=(grid,),
        in_specs=[
            pl.BlockSpec((bb, ED), lambda i: (i, 0)),
            pl.BlockSpec((bb, ED), lambda i: (i, 0)),
            pl.BlockSpec((bb, FD), lambda i: (i, 0)),
            full((ED, HD)),
            full((ED, HD)),
            full((FD, HD)),
            full((1, HD)),
            full((HD, HD)),
            full((1, HD)),
            full((1, HD)),
            full((1, 1)),
        ],
        out_specs=pl.BlockSpec((1, 1, bb), lambda i: (i, 0, 0)),
        out_shape=jax.ShapeDtypeStruct((grid, 1, bb), jnp.float32),
    )(u, p, f, w1a, w1b, w1c, b1, w2, b2, w3r, b3s)


def kernel(user_indices, post_indices, features, user_table, post_table,
           W1, b1, W2, b2, W3, b3):
    uidx = user_indices.astype(jnp.int32)
    pidx = post_indices.astype(jnp.int32)
    utab_rm, ptab_rm = _transpose_tables(user_table.T, post_table.T)
    uout, pout = _make_sc_gather()(uidx, pidx, utab_rm, ptab_rm)
    u = uout.reshape(B, ED)
    p = pout.reshape(B, ED)
    out = _mlp(u, p, features,
               W1[:ED], W1[ED:2 * ED], W1[2 * ED:],
               b1.reshape(1, HD), W2, b2.reshape(1, HD),
               W3.reshape(1, HD), b3.reshape(1, 1), 2048)
    return out.reshape(B)


# speedup vs baseline: 2.4049x; 1.0813x over previous
import functools

import jax
import jax.numpy as jnp
from jax import lax
from jax.experimental import pallas as pl
from jax.experimental.pallas import tpu as pltpu
from jax.experimental.pallas import tpu_sc as plsc

B = 16384
ED = 64
FD = 128
HD = 128
NROWS = 1000000
NC = 2
NS = 16
NW = NC * NS
RPW = B // NW
L = 16
WAVE = 128
TBLK = 16384


TGRID = (NROWS + TBLK - 1) // TBLK
NPACK = TGRID * (TBLK // 2)


def _tr_body(ut_ref, pt_ref, xu_ref, xp_ref):
    tu = ut_ref[...].T
    tp = pt_ref[...].T
    xu_ref[...] = jnp.concatenate([tu[:TBLK // 2], tu[TBLK // 2:]], axis=1)
    xp_ref[...] = jnp.concatenate([tp[:TBLK // 2], tp[TBLK // 2:]], axis=1)


def _transpose_tables(utabT, ptabT):
    return pl.pallas_call(
        _tr_body,
        grid=(TGRID,),
        in_specs=[
            pl.BlockSpec((ED, TBLK), lambda i: (0, i)),
            pl.BlockSpec((ED, TBLK), lambda i: (0, i)),
        ],
        out_specs=[
            pl.BlockSpec((TBLK // 2, 2 * ED), lambda i: (i, 0)),
            pl.BlockSpec((TBLK // 2, 2 * ED), lambda i: (i, 0)),
        ],
        out_shape=[
            jax.ShapeDtypeStruct((NPACK, 2 * ED), jnp.float32),
            jax.ShapeDtypeStruct((NPACK, 2 * ED), jnp.float32),
        ],
    )(utabT, ptabT)


@functools.cache
def _make_sc_gather():
    mesh = plsc.VectorSubcoreMesh(core_axis_name="c", subcore_axis_name="s")

    @functools.partial(
        pl.kernel,
        mesh=mesh,
        out_type=[
            jax.ShapeDtypeStruct((NW, RPW, ED), jnp.float32),
            jax.ShapeDtypeStruct((NW, RPW, ED), jnp.float32),
        ],
        scratch_types=[
            pltpu.VMEM((RPW,), jnp.int32),
            pltpu.VMEM((RPW,), jnp.int32),
            pltpu.VMEM((WAVE, 2 * ED), jnp.float32),
            pltpu.VMEM((RPW, ED), jnp.float32),
            pltpu.SemaphoreType.DMA,
        ],
    )
    def _sc_gather(uidx_hbm, pidx_hbm, utab_hbm, ptab_hbm, uout_hbm, pout_hbm,
                   uidx_v, pidx_v, pair_v, rows_v, sem):
        wid = lax.axis_index("s") * NC + lax.axis_index("c")
        pltpu.sync_copy(uidx_hbm.at[pl.ds(wid * RPW, RPW)], uidx_v)
        pltpu.sync_copy(pidx_hbm.at[pl.ds(wid * RPW, RPW)], pidx_v)

        def one_table(idx_v, tab_hbm, out_hbm):
            for w in range(RPW // WAVE):
                def issue(g, carry):
                    vec = idx_v[pl.ds(w * WAVE + g * L, L)]
                    for lane in range(L):
                        r = vec[lane]
                        k = (r >> 14) * (TBLK // 2) + (r & (TBLK // 2 - 1))
                        pltpu.async_copy(tab_hbm.at[pl.ds(k, 1)],
                                         pair_v.at[pl.ds(g * L + lane, 1)], sem)
                    return carry

                lax.fori_loop(0, WAVE // L, issue, 0)

                def drain(i, carry):
                    pltpu.make_async_copy(tab_hbm.at[pl.ds(0, 1)],
                                          pair_v.at[pl.ds(i, 1)], sem).wait()
                    return carry

                lax.fori_loop(0, WAVE, drain, 0)

                def extract(g, carry):
                    vec = idx_v[pl.ds(w * WAVE + g * L, L)]
                    for lane in range(L):
                        off = ((vec[lane] >> 13) & 1) * ED
                        i = g * L + lane
                        for j in range(ED // L):
                            rows_v[w * WAVE + i, pl.ds(j * L, L)] = (
                                pair_v[i, pl.ds(off + j * L, L)])
                    return carry

                lax.fori_loop(0, WAVE // L, extract, 0)
            pltpu.sync_copy(rows_v, out_hbm.at[wid])

        one_table(uidx_v, utab_hbm, uout_hbm)
        one_table(pidx_v, ptab_hbm, pout_hbm)

    return _sc_gather


def _mlp_body(u_ref, p_ref, f_ref, w1a_ref, w1b_ref, w1c_ref, b1_ref,
              w2_ref, b2_ref, w3_ref, b3_ref, o_ref):
    dot = functools.partial(jnp.dot, preferred_element_type=jnp.float32,
                            precision=lax.Precision.DEFAULT)
    h = dot(u_ref[...], w1a_ref[...])
    h = h + dot(p_ref[...], w1b_ref[...])
    h = h + dot(f_ref[...], w1c_ref[...])
    h = jnp.maximum(h + b1_ref[...], 0.0)
    h = jnp.maximum(dot(h, w2_ref[...]) + b2_ref[...], 0.0)
    v = jnp.sum(h * w3_ref[...], axis=1) + b3_ref[0, 0]
    o_ref[0, 0, :] = 1.0 / (1.0 + jnp.exp(-v))


def _mlp(u, p, f, w1a, w1b, w1c, b1, w2, b2, w3r, b3s, bb):
    grid = B // bb
    full = lambda shape: pl.BlockSpec(shape, lambda i: (0, 0))
    return pl.pallas_call(
        _mlp_body,
        grid=(grid,),
        in_specs=[
            pl.BlockSpec((bb, ED), lambda i: (i, 0)),
            pl.BlockSpec((bb, ED), lambda i: (i, 0)),
            pl.BlockSpec((bb, FD), lambda i: (i, 0)),
            full((ED, HD)),
            full((ED, HD)),
            full((FD, HD)),
            full((1, HD)),
            full((HD, HD)),
            full((1, HD)),
            full((1, HD)),
            full((1, 1)),
        ],
        out_specs=pl.BlockSpec((1, 1, bb), lambda i: (i, 0, 0)),
        out_shape=jax.ShapeDtypeStruct((grid, 1, bb), jnp.float32),
    )(u, p, f, w1a, w1b, w1c, b1, w2, b2, w3r, b3s)


def kernel(user_indices, post_indices, features, user_table, post_table,
           W1, b1, W2, b2, W3, b3):
    uidx = user_indices.astype(jnp.int32)
    pidx = post_indices.astype(jnp.int32)
    utab_rm, ptab_rm = _transpose_tables(user_table.T, post_table.T)
    uout, pout = _make_sc_gather()(uidx, pidx, utab_rm, ptab_rm)
    u = uout.reshape(B, ED)
    p = pout.reshape(B, ED)
    out = _mlp(u, p, features,
               W1[:ED], W1[ED:2 * ED], W1[2 * ED:],
               b1.reshape(1, HD), W2, b2.reshape(1, HD),
               W3.reshape(1, HD), b3.reshape(1, 1), 2048)
    return out.reshape(B)
